# Initial kernel scaffold; baseline (speedup 1.0000x reference)
import jax
import jax.numpy as jnp
from jax.experimental import pallas as pl


def kernel(onehot_enc, edge_attrs, W, att_src, att_dst, W_edge, att_edge, alpha_coef, alpha_mask, edge_indices):
    raise NotImplementedError("write your pallas kernel here")


# speedup vs baseline: 29.8942x; 29.8942x over previous
import functools

import jax
import jax.numpy as jnp
from jax import lax
from jax.experimental import pallas as pl
from jax.experimental.pallas import tpu as pltpu
from jax.experimental.pallas import tpu_sc as plsc

N = 4096
E = 65536
H = 4
C = 16
F_IN = 10

NC = 2
NS = 16
L = 16
ED = E // NS
EO = E // (NC * NS)
NT = N // NS
NROW = EO // 128


def _sc_body(src_hbm, dst_hbm, ea_hbm, oh_hbm, wcat_hbm, aux_hbm, mask_hbm,
             shab, shden, shfin, out_hbm,
             src_v, dst_v, ea_v, oh_v, wcat_v, aux_v, abpart_v, ab_v, ex_v,
             den_v, sum_v, tmp_v, pos2d, wbuf, mbuf, sem):
    c = lax.axis_index("c")
    s = lax.axis_index("s")
    iota = lax.iota(jnp.int32, L)
    zero16 = jnp.zeros((L,), jnp.float32)

    pltpu.sync_copy(src_hbm.at[pl.ds(s * ED, ED)], src_v)
    pltpu.sync_copy(dst_hbm.at[pl.ds(s * ED, ED)], dst_v)
    pltpu.sync_copy(ea_hbm.at[pl.ds(s * ED, ED)], ea_v)
    pltpu.sync_copy(oh_hbm.at[pl.ds(s * NT, NT)], oh_v)
    pltpu.sync_copy(wcat_hbm, wcat_v)
    pltpu.sync_copy(aux_hbm, aux_v)

    wvecs = [wcat_v[pl.ds(i * L, L)] for i in range(8)]

    def _wc(k, j):
        idx = k * (2 * H) + j
        return wvecs[idx // L][idx % L]

    def mm_body(i, carry):
        nodes = i * L + iota
        for j in range(2 * H):
            acc = zero16
            for k in range(F_IN):
                xk = plsc.load_gather(oh_v, [nodes, jnp.full((L,), k, jnp.int32)])
                acc = acc + xk * _wc(k, j)
            plsc.store_scatter(abpart_v, [nodes * (2 * H) + j], acc)
        return carry

    lax.fori_loop(0, NT // L, mm_body, 0)
    pltpu.sync_copy(abpart_v, shab.at[pl.ds(s * NT * 2 * H, NT * 2 * H)])
    plsc.subcore_barrier()
    pltpu.sync_copy(shab, ab_v)

    def z_body(i, carry):
        den_v[pl.ds(i * L, L)] = zero16
        return carry

    lax.fori_loop(0, (N * H) // L, z_body, 0)

    av = aux_v[pl.ds(0, L)]
    we = [av[h] for h in range(H)]
    cf = [av[H + h] for h in range(H)]

    def p1_body(i, carry):
        sl = pl.ds(i * L, L)
        sv = src_v[sl]
        dv = dst_v[sl]
        ev = ea_v[sl]
        eidx = (i * L + iota) * H
        for h in range(H):
            asrc = plsc.load_gather(ab_v, [sv * (2 * H) + h])
            adst = plsc.load_gather(ab_v, [dv * (2 * H) + (H + h)])
            al = asrc + adst + ev * we[h]
            al = jnp.maximum(al, 0.2 * al)
            exv = jnp.exp(al)
            plsc.store_scatter(ex_v, [eidx + h], exv)
            plsc.addupdate_scatter(den_v, [dv * H + h], exv)
        return carry

    lax.fori_loop(0, ED // L, p1_body, 0)

    pltpu.sync_copy(den_v, shden.at[s])
    plsc.subcore_barrier()
    base = s * (N * H // NS)

    def zs_body(i, carry):
        sum_v[pl.ds(i * L, L)] = zero16
        return carry

    lax.fori_loop(0, (N * H // NS) // L, zs_body, 0)
    for t in range(NS):
        pltpu.sync_copy(shden.at[t, pl.ds(base, N * H // NS)], tmp_v)

        def add_body(i, carry):
            sl = pl.ds(i * L, L)
            sum_v[sl] = sum_v[sl] + tmp_v[sl]
            return carry

        lax.fori_loop(0, (N * H // NS) // L, add_body, 0)
    pltpu.sync_copy(sum_v, shfin.at[pl.ds(base, N * H // NS)])
    plsc.subcore_barrier()
    pltpu.sync_copy(shfin, den_v)

    off = c * EO

    def p2_body(i, carry):
        sl = pl.ds(off + i * L, L)
        sv = src_v[sl]
        dv = dst_v[sl]
        posv = dv * N + sv
        r = i // 8
        cb = (i - r * 8) * L
        plsc.store_scatter(pos2d, [jnp.full((L,), r, jnp.int32), cb + iota], posv)
        lidx = (off + i * L + iota) * H
        acc = zero16
        for h in range(H):
            exv = plsc.load_gather(ex_v, [lidx + h])
            dnv = plsc.load_gather(den_v, [dv * H + h])
            acc = acc + cf[h] * (exv / (dnv + 1e-16))
        wbuf[pl.ds(i * L, L)] = acc
        return carry

    lax.fori_loop(0, EO // L, p2_body, 0)

    gathers = [
        pltpu.async_copy(mask_hbm.at[pos2d.at[r]], mbuf.at[pl.ds(r * 128, 128)], sem)
        for r in range(NROW)
    ]
    for g in gathers:
        g.wait()

    def mul_body(i, carry):
        sl = pl.ds(i * L, L)
        wbuf[sl] = wbuf[sl] * mbuf[sl]
        return carry

    lax.fori_loop(0, EO // L, mul_body, 0)

    scatters = [
        pltpu.async_copy(wbuf.at[pl.ds(r * 128, 128)], out_hbm.at[pos2d.at[r]], sem)
        for r in range(NROW)
    ]
    for sc in scatters:
        sc.wait()


_mesh = plsc.VectorSubcoreMesh(core_axis_name="c", subcore_axis_name="s")

_sc_kernel = pl.kernel(
    _sc_body,
    out_type=(),
    mesh=_mesh,
    compiler_params=pltpu.CompilerParams(needs_layout_passes=False),
    scratch_types=[
        pltpu.VMEM((ED,), jnp.int32),
        pltpu.VMEM((ED,), jnp.int32),
        pltpu.VMEM((ED,), jnp.float32),
        pltpu.VMEM((NT, F_IN), jnp.float32),
        pltpu.VMEM((128,), jnp.float32),
        pltpu.VMEM((16,), jnp.float32),
        pltpu.VMEM((NT * 2 * H,), jnp.float32),
        pltpu.VMEM((N * 2 * H,), jnp.float32),
        pltpu.VMEM((ED * H,), jnp.float32),
        pltpu.VMEM((N * H,), jnp.float32),
        pltpu.VMEM((N * H // NS,), jnp.float32),
        pltpu.VMEM((N * H // NS,), jnp.float32),
        pltpu.VMEM((NROW, 128), jnp.int32),
        pltpu.VMEM((EO,), jnp.float32),
        pltpu.VMEM((EO,), jnp.float32),
        pltpu.SemaphoreType.DMA,
    ],
)


@jax.jit
def kernel(onehot_enc, edge_attrs, W, att_src, att_dst, W_edge, att_edge,
           alpha_coef, alpha_mask, edge_indices):
    src = edge_indices[0]
    dst = edge_indices[1]
    ea = edge_attrs[:, 0]
    Wh = W.reshape(F_IN, H, C)
    wcat = jnp.concatenate(
        [jnp.einsum("khc,hc->kh", Wh, att_src[0]),
         jnp.einsum("khc,hc->kh", Wh, att_dst[0])], axis=1)
    wcat = jnp.concatenate([wcat.reshape(-1), jnp.zeros((48,), jnp.float32)])
    we = (W_edge.reshape(H, C) * att_edge[0]).sum(-1)
    cf = jax.nn.softmax(alpha_coef.reshape(H))
    aux = jnp.concatenate([we, cf, jnp.zeros((8,), jnp.float32)])
    mask_flat = alpha_mask.reshape(-1)
    out_ref = jax.new_ref(jnp.zeros((N * N,), jnp.float32))
    shab = jax.new_ref(jnp.zeros((N * 2 * H,), jnp.float32))
    shden = jax.new_ref(jnp.zeros((NS, N * H), jnp.float32))
    shfin = jax.new_ref(jnp.zeros((N * H,), jnp.float32))
    _sc_kernel(src, dst, ea, onehot_enc, wcat, aux, mask_flat,
               shab, shden, shfin, out_ref)
    return out_ref[...].reshape(N, N)
